# baseline (speedup 1.0000x reference)
import functools

import jax
import jax.numpy as jnp
from jax import lax
from jax.experimental import pallas as pl
from jax.experimental.pallas import tpu as pltpu
from jax.experimental.pallas import tpu_sc as plsc

B = 32
D = 1024
E = 128
L = 16
NC = 2
NS = 16
NW = NC * NS
UNROLL = 8
LIP = 0.05
VALID_STEP = 10.0

_mesh = plsc.VectorSubcoreMesh(
    core_axis_name="c", subcore_axis_name="s", num_cores=NC, num_subcores=NS)


def _rsqrt(x):
    i = lax.bitcast_convert_type(x, jnp.int32)
    y = lax.bitcast_convert_type(jnp.int32(0x5F3759DF) - (i >> 1), jnp.float32)
    for _ in range(3):
        y = y * (1.5 - 0.5 * x * y * y)
    return y


def _sc_body(inp_hbm, out_hbm, part_hbm, inp_v, out_v, part_v):
    b = lax.axis_index("s") * NC + lax.axis_index("c")
    lane = lax.iota(jnp.int32, L)

    pltpu.sync_copy(inp_hbm.at[pl.ds(b * (4 * D), 4 * D)], inp_v)
    pltpu.sync_copy(out_hbm.at[pl.ds(b * (4 * E), 4 * E)], out_v)

    def hr_body(c, acc):
        for u in range(UNROLL):
            h = out_v[pl.ds((c * UNROLL + u) * L, L)]
            acc = acc + h * h
        return acc
    ss0 = jnp.sum(lax.fori_loop(0, E // (L * UNROLL), hr_body,
                                jnp.zeros((L,), jnp.float32)))
    r0 = _rsqrt(ss0)

    partial = jnp.float32(0.0)
    for j in range(1, 4):
        def inp_body(c, carry):
            acc, mx = carry
            for u in range(UNROLL):
                o = (c * UNROLL + u) * L
                d = inp_v[pl.ds(j * D + o, L)] - inp_v[pl.ds(o, L)]
                acc = acc + d * d
                mx = jnp.maximum(mx, jnp.abs(d))
            return acc, mx
        acc, mx = lax.fori_loop(
            0, D // (L * UNROLL), inp_body,
            (jnp.zeros((L,), jnp.float32), jnp.zeros((L,), jnp.float32)))
        ss_inp = jnp.sum(acc)
        same = jnp.max(mx) == 0.0

        def out_body(c, carry):
            aj, ad = carry
            for u in range(UNROLL):
                o = (c * UNROLL + u) * L
                h = out_v[pl.ds(o, L)]
                lr = out_v[pl.ds(j * E + o, L)]
                aj = aj + lr * lr
                ad = ad + h * lr
            return aj, ad
        aj, ad = lax.fori_loop(
            0, E // (L * UNROLL), out_body,
            (jnp.zeros((L,), jnp.float32), jnp.zeros((L,), jnp.float32)))
        cos = jnp.sum(ad) * r0 * _rsqrt(jnp.sum(aj))
        ratio = (1.0 - cos) * _rsqrt(ss_inp)
        pen = jnp.maximum(ratio - LIP, 0.0)
        pen = jnp.where(same, 0.0, pen)
        partial = partial + pen

    part_v[...] = jnp.where(lane == 0, partial, 0.0)
    pltpu.sync_copy(part_v, part_hbm.at[b])


_sc_call = functools.partial(
    pl.kernel,
    out_type=jax.ShapeDtypeStruct((NW, L), jnp.float32),
    mesh=_mesh,
    compiler_params=pltpu.CompilerParams(needs_layout_passes=False),
    scratch_types=[
        pltpu.VMEM((4 * D,), jnp.float32),
        pltpu.VMEM((4 * E,), jnp.float32),
        pltpu.VMEM((L,), jnp.float32),
    ],
)(_sc_body)


def _fin_body(part_ref, out_ref, labels_ref, o_ref):
    partials = part_ref[...]
    labels = labels_ref[...]
    outv = out_ref[...]
    loss = jnp.sum(partials) / jnp.float32(B * 3)

    match = labels[:, 0] == labels[0, 0]
    k = jnp.sum(match.astype(jnp.int32))
    nw = jnp.minimum(k, 4)
    row = jax.lax.broadcasted_iota(jnp.int32, (B, B), 0)
    col = jax.lax.broadcasted_iota(jnp.int32, (B, B), 1)
    after = jnp.sum(jnp.where((col > row) & match[None, :], 1, 0), axis=1)
    survive = match & (after <= 3)
    rowsum = jnp.sum(outv[:, 0, :], axis=-1)
    mem_sum = jnp.sum(jnp.where(survive, rowsum, 0.0))
    val_sum = VALID_STEP * nw.astype(jnp.float32) - 4.0

    total = loss + 0.0 * mem_sum + 0.0 * val_sum
    o_ref[...] = jnp.broadcast_to(total, (1, 1))


def kernel(inp, out, labels, memory_bank_HR, validness):
    del memory_bank_HR, validness
    partials = _sc_call(inp.reshape(B * 4 * D), out.reshape(B * 4 * E))
    res = pl.pallas_call(
        _fin_body,
        out_shape=jax.ShapeDtypeStruct((1, 1), jnp.float32),
    )(partials, out, labels.reshape(B, 1).astype(jnp.int32))
    return res[0, 0]


# speedup vs baseline: 1.0039x; 1.0039x over previous
import functools

import jax
import jax.numpy as jnp
from jax import lax
from jax.experimental import pallas as pl
from jax.experimental.pallas import tpu as pltpu
from jax.experimental.pallas import tpu_sc as plsc

B = 32
D = 1024
E = 128
L = 16
NC = 2
NS = 16
NW = NC * NS
UNROLL = 4
LIP = 0.05
VALID_STEP = 10.0

_mesh = plsc.VectorSubcoreMesh(
    core_axis_name="c", subcore_axis_name="s", num_cores=NC, num_subcores=NS)


def _rsqrt(x):
    i = lax.bitcast_convert_type(x, jnp.int32)
    y = lax.bitcast_convert_type(jnp.int32(0x5F3759DF) - (i >> 1), jnp.float32)
    for _ in range(3):
        y = y * (1.5 - 0.5 * x * y * y)
    return y


def _sc_body(inp_hbm, out_hbm, part_hbm, inp_v, out_v, part_v):
    b = lax.axis_index("s") * NC + lax.axis_index("c")
    lane = lax.iota(jnp.int32, L)

    pltpu.sync_copy(inp_hbm.at[pl.ds(b * (4 * D), 4 * D)], inp_v)
    pltpu.sync_copy(out_hbm.at[pl.ds(b * (4 * E), 4 * E)], out_v)

    def hr_body(c, acc):
        for u in range(UNROLL):
            h = out_v[pl.ds((c * UNROLL + u) * L, L)]
            acc = acc + h * h
        return acc
    ss0 = jnp.sum(lax.fori_loop(0, E // (L * UNROLL), hr_body,
                                jnp.zeros((L,), jnp.float32)))
    r0 = _rsqrt(ss0)

    partial = jnp.float32(0.0)
    for j in range(1, 4):
        def inp_body(c, carry):
            acc, mx = carry
            for u in range(UNROLL):
                o = (c * UNROLL + u) * L
                d = inp_v[pl.ds(j * D + o, L)] - inp_v[pl.ds(o, L)]
                acc = acc + d * d
                mx = jnp.maximum(mx, jnp.abs(d))
            return acc, mx
        acc, mx = lax.fori_loop(
            0, D // (L * UNROLL), inp_body,
            (jnp.zeros((L,), jnp.float32), jnp.zeros((L,), jnp.float32)))
        ss_inp = jnp.sum(acc)
        same = jnp.max(mx) == 0.0

        def out_body(c, carry):
            aj, ad = carry
            for u in range(UNROLL):
                o = (c * UNROLL + u) * L
                h = out_v[pl.ds(o, L)]
                lr = out_v[pl.ds(j * E + o, L)]
                aj = aj + lr * lr
                ad = ad + h * lr
            return aj, ad
        aj, ad = lax.fori_loop(
            0, E // (L * UNROLL), out_body,
            (jnp.zeros((L,), jnp.float32), jnp.zeros((L,), jnp.float32)))
        cos = jnp.sum(ad) * r0 * _rsqrt(jnp.sum(aj))
        ratio = (1.0 - cos) * _rsqrt(ss_inp)
        pen = jnp.maximum(ratio - LIP, 0.0)
        pen = jnp.where(same, 0.0, pen)
        partial = partial + pen

    part_v[...] = jnp.where(lane == 0, partial, 0.0)
    pltpu.sync_copy(part_v, part_hbm.at[b])


_sc_call = functools.partial(
    pl.kernel,
    out_type=jax.ShapeDtypeStruct((NW, L), jnp.float32),
    mesh=_mesh,
    compiler_params=pltpu.CompilerParams(needs_layout_passes=False),
    scratch_types=[
        pltpu.VMEM((4 * D,), jnp.float32),
        pltpu.VMEM((4 * E,), jnp.float32),
        pltpu.VMEM((L,), jnp.float32),
    ],
)(_sc_body)


def _fin_body(part_ref, out_ref, labels_ref, o_ref):
    partials = part_ref[...]
    labels = labels_ref[...]
    outv = out_ref[...]
    loss = jnp.sum(partials) / jnp.float32(B * 3)

    match = labels[:, 0] == labels[0, 0]
    k = jnp.sum(match.astype(jnp.int32))
    nw = jnp.minimum(k, 4)
    row = jax.lax.broadcasted_iota(jnp.int32, (B, B), 0)
    col = jax.lax.broadcasted_iota(jnp.int32, (B, B), 1)
    after = jnp.sum(jnp.where((col > row) & match[None, :], 1, 0), axis=1)
    survive = match & (after <= 3)
    rowsum = jnp.sum(outv[:, 0, :], axis=-1)
    mem_sum = jnp.sum(jnp.where(survive, rowsum, 0.0))
    val_sum = VALID_STEP * nw.astype(jnp.float32) - 4.0

    total = loss + 0.0 * mem_sum + 0.0 * val_sum
    o_ref[...] = jnp.broadcast_to(total, (1, 1))


def kernel(inp, out, labels, memory_bank_HR, validness):
    del memory_bank_HR, validness
    partials = _sc_call(inp.reshape(B * 4 * D), out.reshape(B * 4 * E))
    res = pl.pallas_call(
        _fin_body,
        out_shape=jax.ShapeDtypeStruct((1, 1), jnp.float32),
    )(partials, out, labels.reshape(B, 1).astype(jnp.int32))
    return res[0, 0]


# speedup vs baseline: 1.0114x; 1.0075x over previous
import functools

import jax
import jax.numpy as jnp
from jax import lax
from jax.experimental import pallas as pl
from jax.experimental.pallas import tpu as pltpu
from jax.experimental.pallas import tpu_sc as plsc

B = 32
D = 1024
E = 128
L = 16
NC = 2
NS = 16
NW = NC * NS
UNROLL = 4
LIP = 0.05
VALID_STEP = 10.0

_mesh = plsc.VectorSubcoreMesh(
    core_axis_name="c", subcore_axis_name="s", num_cores=NC, num_subcores=NS)


def _rsqrt(x):
    i = lax.bitcast_convert_type(x, jnp.int32)
    y = lax.bitcast_convert_type(jnp.int32(0x5F3759DF) - (i >> 1), jnp.float32)
    for _ in range(3):
        y = y * (1.5 - 0.5 * x * y * y)
    return y


def _sc_body(inp_hbm, out_hbm, part_hbm, inp_v, out_v, part_v):
    b = lax.axis_index("s") * NC + lax.axis_index("c")
    lane = lax.iota(jnp.int32, L)

    pltpu.sync_copy(inp_hbm.at[pl.ds(b * (4 * D), 4 * D)], inp_v)
    pltpu.sync_copy(out_hbm.at[pl.ds(b * (4 * E), 4 * E)], out_v)

    def hr_body(c, carry):
        acc, asum = carry
        for u in range(UNROLL):
            h = out_v[pl.ds((c * UNROLL + u) * L, L)]
            acc = acc + h * h
            asum = asum + h
        return acc, asum
    acc0, asum0 = lax.fori_loop(
        0, E // (L * UNROLL), hr_body,
        (jnp.zeros((L,), jnp.float32), jnp.zeros((L,), jnp.float32)))
    ss0 = jnp.sum(acc0)
    rowsum = jnp.sum(asum0)
    r0 = _rsqrt(ss0)

    partial = jnp.float32(0.0)
    for j in range(1, 4):
        def inp_body(c, carry):
            acc, mx = carry
            for u in range(UNROLL):
                o = (c * UNROLL + u) * L
                d = inp_v[pl.ds(j * D + o, L)] - inp_v[pl.ds(o, L)]
                acc = acc + d * d
                mx = jnp.maximum(mx, jnp.abs(d))
            return acc, mx
        acc, mx = lax.fori_loop(
            0, D // (L * UNROLL), inp_body,
            (jnp.zeros((L,), jnp.float32), jnp.zeros((L,), jnp.float32)))
        ss_inp = jnp.sum(acc)
        same = jnp.max(mx) == 0.0

        def out_body(c, carry):
            aj, ad = carry
            for u in range(UNROLL):
                o = (c * UNROLL + u) * L
                h = out_v[pl.ds(o, L)]
                lr = out_v[pl.ds(j * E + o, L)]
                aj = aj + lr * lr
                ad = ad + h * lr
            return aj, ad
        aj, ad = lax.fori_loop(
            0, E // (L * UNROLL), out_body,
            (jnp.zeros((L,), jnp.float32), jnp.zeros((L,), jnp.float32)))
        cos = jnp.sum(ad) * r0 * _rsqrt(jnp.sum(aj))
        ratio = (1.0 - cos) * _rsqrt(ss_inp)
        pen = jnp.maximum(ratio - LIP, 0.0)
        pen = jnp.where(same, 0.0, pen)
        partial = partial + pen

    part_v[...] = jnp.where(lane == 0, partial,
                            jnp.where(lane == 1, rowsum, 0.0))
    pltpu.sync_copy(part_v, part_hbm.at[b])


_sc_call = functools.partial(
    pl.kernel,
    out_type=jax.ShapeDtypeStruct((NW, L), jnp.float32),
    mesh=_mesh,
    compiler_params=pltpu.CompilerParams(needs_layout_passes=False),
    scratch_types=[
        pltpu.VMEM((4 * D,), jnp.float32),
        pltpu.VMEM((4 * E,), jnp.float32),
        pltpu.VMEM((L,), jnp.float32),
    ],
)(_sc_body)


def _fin_body(part_ref, labels_ref, o_ref):
    partials = part_ref[...]
    labels = labels_ref[...]
    lane = jax.lax.broadcasted_iota(jnp.int32, (NW, L), 1)
    loss = jnp.sum(jnp.where(lane == 0, partials, 0.0)) / jnp.float32(B * 3)

    match = labels[:, 0] == labels[0, 0]
    k = jnp.sum(match.astype(jnp.int32))
    nw = jnp.minimum(k, 4)
    row = jax.lax.broadcasted_iota(jnp.int32, (B, B), 0)
    col = jax.lax.broadcasted_iota(jnp.int32, (B, B), 1)
    after = jnp.sum(jnp.where((col > row) & match[None, :], 1, 0), axis=1)
    survive = match & (after <= 3)
    rowsum = partials[:, 1]
    mem_sum = jnp.sum(jnp.where(survive, rowsum, 0.0))
    val_sum = VALID_STEP * nw.astype(jnp.float32) - 4.0

    total = loss + 0.0 * mem_sum + 0.0 * val_sum
    o_ref[...] = jnp.broadcast_to(total, (1, 1))


def kernel(inp, out, labels, memory_bank_HR, validness):
    del memory_bank_HR, validness
    partials = _sc_call(inp.reshape(B * 4 * D), out.reshape(B * 4 * E))
    res = pl.pallas_call(
        _fin_body,
        out_shape=jax.ShapeDtypeStruct((1, 1), jnp.float32),
    )(partials, labels.reshape(B, 1).astype(jnp.int32))
    return res[0, 0]
